# baseline (speedup 1.0000x reference)
import functools

import jax
import jax.numpy as jnp
from jax import lax
from jax.experimental import pallas as pl
from jax.experimental.pallas import tpu as pltpu
from jax.experimental.pallas import tpu_sc as plsc

_EMBED = 64
_NC = 2
_NS = 16
_NW = _NC * _NS
_NBUF = 4
_NWB = 2


@functools.partial(jax.jit, static_argnames=("batch", "hist"))
def _lookup(idx, tablep, *, batch, hist):
    bat_w = batch // _NW
    assert bat_w % _NBUF == 0 and _NBUF % _NWB == 0
    mesh = plsc.VectorSubcoreMesh(core_axis_name="c", subcore_axis_name="s")
    ngr = (hist + 15) // 16

    @functools.partial(
        pl.kernel,
        out_type=jax.ShapeDtypeStruct((batch, hist, _EMBED), jnp.float32),
        mesh=mesh,
        scratch_types=[
            pltpu.VMEM((bat_w, hist), jnp.int32),
            pltpu.VMEM((bat_w, hist), jnp.int32),
            pltpu.VMEM((_NBUF, hist, 2 * _EMBED), jnp.float32),
            pltpu.VMEM((_NWB, hist, _EMBED), jnp.float32),
            [pltpu.SemaphoreType.DMA] * _NBUF,
            [pltpu.SemaphoreType.DMA] * _NWB,
        ],
        compiler_params=pltpu.CompilerParams(needs_layout_passes=False),
    )
    def body(idx_hbm, tab_hbm, out_hbm, idx_v, pidx_v, pair, outb, sem_g, sem_w):
        wid = lax.axis_index("s") * _NC + lax.axis_index("c")
        bi0 = wid * bat_w
        lanes = lax.iota(jnp.int32, 16)
        pltpu.sync_copy(idx_hbm.at[pl.ds(bi0, bat_w)], idx_v)

        @pl.loop(0, bat_w)
        def _(r):
            rv = jnp.full((16,), r, jnp.int32)
            for j in range(ngr):
                cols = j * 16 + lanes
                m = cols < hist
                cc = jnp.where(m, cols, 0)
                v = plsc.load_gather(idx_v, [rv, cc], mask=m)
                plsc.store_scatter(pidx_v, [rv, cc], v >> 1, mask=m)

        def gather(g, b):
            return pltpu.make_async_copy(
                tab_hbm.at[pidx_v.at[g]], pair.at[b], sem_g[b])

        def write(g, w):
            return pltpu.make_async_copy(
                outb.at[w], out_hbm.at[bi0 + g], sem_w[w])

        def extract(g, b, w):
            gv = jnp.full((16,), g, jnp.int32)
            for j in range(ngr):
                rows = j * 16 + lanes
                m = rows < hist
                rr = jnp.where(m, rows, 0)
                hv = (plsc.load_gather(idx_v, [gv, rr], mask=m) & 1) * _EMBED

                @pl.loop(0, _EMBED, unroll=8)
                def _(c0):
                    v = plsc.load_gather(pair.at[b], [rr, hv + c0], mask=m)
                    plsc.store_scatter(
                        outb.at[w], [rr, jnp.full((16,), c0, jnp.int32)], v,
                        mask=m)

        def visit(g, b, need_wwait):
            gather(g, b).wait()
            w = b % _NWB
            if need_wwait:
                write(g - _NWB, w).wait()
            extract(g, b, w)
            write(g, w).start()

        for b in range(_NBUF):
            gather(b, b).start()

        for b in range(_NBUF):
            visit(b, b, b >= _NWB)
            gather(b + _NBUF, b).start()

        @pl.loop(_NBUF, bat_w - _NBUF, step=_NBUF)
        def _(g0):
            for b in range(_NBUF):
                g = g0 + b
                visit(g, b, True)
                gather(g + _NBUF, b).start()

        for b in range(_NBUF):
            visit(bat_w - _NBUF + b, b, True)
        for g in range(bat_w - _NWB, bat_w):
            write(g, g % _NWB).wait()

    return body(idx, tablep)


def kernel(input, table):
    batch, hist = input.shape
    vocab = table.shape[0]
    tablep = table.reshape(vocab // 2, 2 * _EMBED)
    return _lookup(input.astype(jnp.int32), tablep, batch=batch, hist=hist)


# speedup vs baseline: 1.6707x; 1.6707x over previous
import functools

import jax
import jax.numpy as jnp
from jax import lax
from jax.experimental import pallas as pl
from jax.experimental.pallas import tpu as pltpu
from jax.experimental.pallas import tpu_sc as plsc

_EMBED = 64
_NC = 2
_NS = 16
_NW = _NC * _NS
_NBUF = 4
_NWB = 2


@functools.partial(jax.jit, static_argnames=("batch", "hist"))
def _lookup(idx, tablep, *, batch, hist):
    bat_w = batch // _NW
    assert bat_w % _NBUF == 0 and _NBUF % _NWB == 0
    mesh = plsc.VectorSubcoreMesh(core_axis_name="c", subcore_axis_name="s")
    ngr = (hist + 15) // 16

    @functools.partial(
        pl.kernel,
        out_type=jax.ShapeDtypeStruct((batch, hist, _EMBED), jnp.float32),
        mesh=mesh,
        scratch_types=[
            pltpu.VMEM((bat_w, hist), jnp.int32),
            pltpu.VMEM((_NBUF, hist, 2 * _EMBED), jnp.float32),
            pltpu.VMEM((_NWB, hist, _EMBED), jnp.float32),
            [pltpu.SemaphoreType.DMA] * _NBUF,
            [pltpu.SemaphoreType.DMA] * _NWB,
        ],
        compiler_params=pltpu.CompilerParams(needs_layout_passes=False),
    )
    def body(idx_hbm, tab_hbm, out_hbm, idx_v, pair, outb, sem_g, sem_w):
        wid = lax.axis_index("s") * _NC + lax.axis_index("c")
        bi0 = wid * bat_w
        pltpu.sync_copy(idx_hbm.at[pl.ds(bi0, bat_w)], idx_v)

        def gather(g, b):
            return pltpu.make_async_copy(
                tab_hbm.at[idx_v.at[g]], pair.at[b], sem_g[b])

        def write(g, w):
            return pltpu.make_async_copy(
                outb.at[w], out_hbm.at[bi0 + g], sem_w[w])

        def extract(g, b, w):
            del g

            @pl.loop(0, hist, unroll=5)
            def _(r):
                for c0 in range(0, _EMBED, 16):
                    outb.at[w][r, pl.ds(c0, 16)] = pair.at[b][r, pl.ds(c0, 16)]

        def visit(g, b, need_wwait):
            gather(g, b).wait()
            w = b % _NWB
            if need_wwait:
                write(g - _NWB, w).wait()
            extract(g, b, w)
            write(g, w).start()

        for b in range(_NBUF):
            gather(b, b).start()

        for b in range(_NBUF):
            visit(b, b, b >= _NWB)
            gather(b + _NBUF, b).start()

        @pl.loop(_NBUF, bat_w - _NBUF, step=_NBUF)
        def _(g0):
            for b in range(_NBUF):
                g = g0 + b
                visit(g, b, True)
                gather(g + _NBUF, b).start()

        for b in range(_NBUF):
            visit(bat_w - _NBUF + b, b, True)
        for g in range(bat_w - _NWB, bat_w):
            write(g, g % _NWB).wait()

    return body(idx, tablep)


def kernel(input, table):
    batch, hist = input.shape
    vocab = table.shape[0]
    del vocab
    tablep = jnp.pad(table, ((0, 0), (0, _EMBED)))
    return _lookup(input.astype(jnp.int32), tablep, batch=batch, hist=hist)


# speedup vs baseline: 1.6873x; 1.0099x over previous
import functools

import jax
import jax.numpy as jnp
from jax import lax
from jax.experimental import pallas as pl
from jax.experimental.pallas import tpu as pltpu
from jax.experimental.pallas import tpu_sc as plsc

_EMBED = 64
_NC = 2
_NS = 16
_NW = _NC * _NS
_NBUF = 8
_NWB = 2


@functools.partial(jax.jit, static_argnames=("batch", "hist"))
def _lookup(idx, tablep, *, batch, hist):
    bat_w = batch // _NW
    assert bat_w % _NBUF == 0 and _NBUF % _NWB == 0
    mesh = plsc.VectorSubcoreMesh(core_axis_name="c", subcore_axis_name="s")
    ngr = (hist + 15) // 16

    @functools.partial(
        pl.kernel,
        out_type=jax.ShapeDtypeStruct((batch, hist, _EMBED), jnp.float32),
        mesh=mesh,
        scratch_types=[
            pltpu.VMEM((bat_w, hist), jnp.int32),
            pltpu.VMEM((_NBUF, hist, 2 * _EMBED), jnp.float32),
            pltpu.VMEM((_NWB, hist, _EMBED), jnp.float32),
            [pltpu.SemaphoreType.DMA] * _NBUF,
            [pltpu.SemaphoreType.DMA] * _NWB,
        ],
        compiler_params=pltpu.CompilerParams(needs_layout_passes=False),
    )
    def body(idx_hbm, tab_hbm, out_hbm, idx_v, pair, outb, sem_g, sem_w):
        wid = lax.axis_index("s") * _NC + lax.axis_index("c")
        bi0 = wid * bat_w
        pltpu.sync_copy(idx_hbm.at[pl.ds(bi0, bat_w)], idx_v)

        def gather(g, b):
            return pltpu.make_async_copy(
                tab_hbm.at[idx_v.at[g]], pair.at[b], sem_g[b])

        def write(g, w):
            return pltpu.make_async_copy(
                outb.at[w], out_hbm.at[bi0 + g], sem_w[w])

        def extract(g, b, w):
            del g

            @pl.loop(0, hist, unroll=5)
            def _(r):
                for c0 in range(0, _EMBED, 16):
                    outb.at[w][r, pl.ds(c0, 16)] = pair.at[b][r, pl.ds(c0, 16)]

        def visit(g, b, need_wwait):
            gather(g, b).wait()
            w = b % _NWB
            if need_wwait:
                write(g - _NWB, w).wait()
            extract(g, b, w)
            write(g, w).start()

        for b in range(_NBUF):
            gather(b, b).start()

        for b in range(_NBUF):
            visit(b, b, b >= _NWB)
            gather(b + _NBUF, b).start()

        @pl.loop(_NBUF, bat_w - _NBUF, step=_NBUF)
        def _(g0):
            for b in range(_NBUF):
                g = g0 + b
                visit(g, b, True)
                gather(g + _NBUF, b).start()

        for b in range(_NBUF):
            visit(bat_w - _NBUF + b, b, True)
        for g in range(bat_w - _NWB, bat_w):
            write(g, g % _NWB).wait()

    return body(idx, tablep)


def kernel(input, table):
    batch, hist = input.shape
    vocab = table.shape[0]
    del vocab
    tablep = jnp.pad(table, ((0, 0), (0, _EMBED)))
    return _lookup(input.astype(jnp.int32), tablep, batch=batch, hist=hist)
